# baseline (speedup 1.0000x reference)
import functools

import jax
import jax.numpy as jnp
from jax import lax
from jax.experimental import pallas as pl
from jax.experimental.pallas import tpu as pltpu
from jax.experimental.pallas import tpu_sc as plsc

VOCAB = 100000
HIDDEN = 1024
MAX_POS = 2048
BATCH = 4
SEQ = 2048

_INFO = plsc.get_sparse_core_info()
NC = _INFO.num_cores
NS = _INFO.num_subcores
LANES = _INFO.num_lanes
NW = NC * NS

TOKENS = BATCH * SEQ
TOK_PER_W = TOKENS // NW
CHUNK = 32
NCHUNK = TOK_PER_W // CHUNK
GROUPS = HIDDEN // LANES


def _make_kernel():
    mesh = plsc.VectorSubcoreMesh(core_axis_name="c", subcore_axis_name="s")

    @functools.partial(
        pl.kernel,
        mesh=mesh,
        out_type=jax.ShapeDtypeStruct((TOKENS, HIDDEN), jnp.float32),
        scratch_types=[
            pltpu.VMEM((CHUNK,), jnp.int32),
            pltpu.VMEM((CHUNK,), jnp.int32),
            pltpu.VMEM((CHUNK, HIDDEN), jnp.float32),
            pltpu.VMEM((CHUNK, HIDDEN), jnp.float32),
            pltpu.SemaphoreType.DMA,
            pltpu.SemaphoreType.DMA,
        ],
    )
    def emb_kernel(tok_ids, pos_ids, tok_tab, pos_tab, out,
                   tok_idx_v, pos_idx_v, tok_buf, pos_buf, sem1, sem2):
        wid = lax.axis_index("s") * NC + lax.axis_index("c")
        base = wid * TOK_PER_W

        def chunk_body(j, carry):
            off = base + j * CHUNK
            pltpu.sync_copy(tok_ids.at[pl.ds(off, CHUNK)], tok_idx_v)
            pltpu.sync_copy(pos_ids.at[pl.ds(off, CHUNK)], pos_idx_v)
            cp1 = pltpu.async_copy(tok_tab.at[tok_idx_v], tok_buf, sem1)
            cp2 = pltpu.async_copy(pos_tab.at[pos_idx_v], pos_buf, sem2)
            cp1.wait()
            cp2.wait()

            def row_body(r, carry2):
                def grp_body(g, carry3):
                    sl = pl.ds(g * LANES, LANES)
                    tok_buf[r, sl] = tok_buf[r, sl] + pos_buf[r, sl]
                    return carry3
                return lax.fori_loop(0, GROUPS, grp_body, carry2)

            lax.fori_loop(0, CHUNK, row_body, 0)
            pltpu.sync_copy(tok_buf, out.at[pl.ds(off, CHUNK)])
            return carry

        lax.fori_loop(0, NCHUNK, chunk_body, 0)

    return emb_kernel


_EMB_KERNEL = _make_kernel()


def kernel(input_ids, position_ids, token_table, pos_table):
    tok_ids = input_ids.reshape(-1).astype(jnp.int32)
    pos_ids = position_ids.reshape(-1).astype(jnp.int32)
    out = _EMB_KERNEL(tok_ids, pos_ids, token_table, pos_table)
    return out.reshape(BATCH, SEQ, HIDDEN)


# speedup vs baseline: 1.5934x; 1.5934x over previous
import functools

import jax
import jax.numpy as jnp
from jax import lax
from jax.experimental import pallas as pl
from jax.experimental.pallas import tpu as pltpu
from jax.experimental.pallas import tpu_sc as plsc

VOCAB = 100000
HIDDEN = 1024
MAX_POS = 2048
BATCH = 4
SEQ = 2048

_INFO = plsc.get_sparse_core_info()
NC = _INFO.num_cores
NS = _INFO.num_subcores
LANES = _INFO.num_lanes
NW = NC * NS

TOKENS = BATCH * SEQ
TOK_PER_W = TOKENS // NW
CHUNK = 16
NCHUNK = TOK_PER_W // CHUNK
NBUF = 2
GROUPS = HIDDEN // LANES


def _make_kernel():
    mesh = plsc.VectorSubcoreMesh(core_axis_name="c", subcore_axis_name="s")

    @functools.partial(
        pl.kernel,
        mesh=mesh,
        out_type=jax.ShapeDtypeStruct((TOKENS, HIDDEN), jnp.float32),
        scratch_types=[
            pltpu.VMEM((NBUF, CHUNK), jnp.int32),
            pltpu.VMEM((NBUF, CHUNK), jnp.int32),
            pltpu.VMEM((NBUF, CHUNK, HIDDEN), jnp.float32),
            pltpu.VMEM((NBUF, CHUNK, HIDDEN), jnp.float32),
        ] + [pltpu.SemaphoreType.DMA] * (3 * NBUF),
    )
    def emb_kernel(tok_ids, pos_ids, tok_tab, pos_tab, out,
                   idx_t, idx_p, tok_buf, pos_buf, *sems):
        sem_t = sems[0:NBUF]
        sem_p = sems[NBUF:2 * NBUF]
        sem_o = sems[2 * NBUF:3 * NBUF]
        wid = lax.axis_index("s") * NC + lax.axis_index("c")
        base = wid * TOK_PER_W

        def issue(j, b):
            off = base + j * CHUNK
            pltpu.sync_copy(tok_ids.at[pl.ds(off, CHUNK)], idx_t.at[b])
            pltpu.sync_copy(pos_ids.at[pl.ds(off, CHUNK)], idx_p.at[b])
            pltpu.async_copy(tok_tab.at[idx_t.at[b]], tok_buf.at[b], sem_t[b])
            pltpu.async_copy(pos_tab.at[idx_p.at[b]], pos_buf.at[b], sem_p[b])

        for b in range(NBUF):
            issue(b, b)

        def outer(jj, carry):
            for b in range(NBUF):
                j = jj * NBUF + b
                off = base + j * CHUNK
                pltpu.make_async_copy(
                    tok_tab.at[idx_t.at[b]], tok_buf.at[b], sem_t[b]).wait()
                pltpu.make_async_copy(
                    pos_tab.at[idx_p.at[b]], pos_buf.at[b], sem_p[b]).wait()

                def row_body(r, c2, _b=b):
                    for g in range(GROUPS):
                        sl = pl.ds(g * LANES, LANES)
                        tok_buf[_b, r, sl] = tok_buf[_b, r, sl] + pos_buf[_b, r, sl]
                    return c2

                lax.fori_loop(0, CHUNK, row_body, 0)
                pltpu.async_copy(tok_buf.at[b], out.at[pl.ds(off, CHUNK)],
                                 sem_o[b])

                nj = j + NBUF

                @pl.when(nj < NCHUNK)
                def _(_b=b, _nj=nj):
                    pltpu.make_async_copy(
                        tok_buf.at[_b],
                        out.at[pl.ds(base + (_nj - NBUF) * CHUNK, CHUNK)],
                        sem_o[_b]).wait()
                    issue(_nj, _b)
            return carry

        lax.fori_loop(0, NCHUNK // NBUF, outer, 0)

        for b in range(NBUF):
            j = NCHUNK - NBUF + b
            pltpu.make_async_copy(
                tok_buf.at[b], out.at[pl.ds(base + j * CHUNK, CHUNK)],
                sem_o[b]).wait()

    return emb_kernel


_EMB_KERNEL = _make_kernel()


def kernel(input_ids, position_ids, token_table, pos_table):
    tok_ids = input_ids.reshape(-1).astype(jnp.int32)
    pos_ids = position_ids.reshape(-1).astype(jnp.int32)
    out = _EMB_KERNEL(tok_ids, pos_ids, token_table, pos_table)
    return out.reshape(BATCH, SEQ, HIDDEN)


# speedup vs baseline: 1.8313x; 1.1493x over previous
import functools

import jax
import jax.numpy as jnp
from jax import lax
from jax.experimental import pallas as pl
from jax.experimental.pallas import tpu as pltpu
from jax.experimental.pallas import tpu_sc as plsc

VOCAB = 100000
HIDDEN = 1024
MAX_POS = 2048
BATCH = 4
SEQ = 2048

_INFO = plsc.get_sparse_core_info()
NC = _INFO.num_cores
NS = _INFO.num_subcores
LANES = _INFO.num_lanes
NW = NC * NS

TOKENS = BATCH * SEQ
TOK_PER_W = TOKENS // NW
CHUNK = 16
NCHUNK = TOK_PER_W // CHUNK
NBUF = 2
GROUPS = HIDDEN // LANES


def _make_kernel():
    mesh = plsc.VectorSubcoreMesh(core_axis_name="c", subcore_axis_name="s")

    @functools.partial(
        pl.kernel,
        mesh=mesh,
        out_type=jax.ShapeDtypeStruct((TOKENS, HIDDEN), jnp.float32),
        scratch_types=[
            pltpu.VMEM((TOK_PER_W,), jnp.int32),
            pltpu.VMEM((TOK_PER_W,), jnp.int32),
            pltpu.VMEM((NBUF, CHUNK, HIDDEN), jnp.float32),
            pltpu.VMEM((NBUF, CHUNK, HIDDEN), jnp.float32),
        ] + [pltpu.SemaphoreType.DMA] * (3 * NBUF),
    )
    def emb_kernel(tok_ids, pos_ids, tok_tab, pos_tab, out,
                   idx_t, idx_p, tok_buf, pos_buf, *sems):
        sem_t = sems[0:NBUF]
        sem_p = sems[NBUF:2 * NBUF]
        sem_o = sems[2 * NBUF:3 * NBUF]
        wid = lax.axis_index("s") * NC + lax.axis_index("c")
        base = wid * TOK_PER_W

        pltpu.sync_copy(tok_ids.at[pl.ds(base, TOK_PER_W)], idx_t)
        pltpu.sync_copy(pos_ids.at[pl.ds(base, TOK_PER_W)], idx_p)

        def issue(j, b):
            isl = pl.ds(j * CHUNK, CHUNK)
            pltpu.async_copy(tok_tab.at[idx_t.at[isl]], tok_buf.at[b], sem_t[b])
            pltpu.async_copy(pos_tab.at[idx_p.at[isl]], pos_buf.at[b], sem_p[b])

        for b in range(NBUF):
            issue(b, b)

        def outer(jj, carry):
            for b in range(NBUF):
                j = jj * NBUF + b
                off = base + j * CHUNK
                isl = pl.ds(j * CHUNK, CHUNK)
                pltpu.make_async_copy(
                    tok_tab.at[idx_t.at[isl]], tok_buf.at[b], sem_t[b]).wait()
                pltpu.make_async_copy(
                    pos_tab.at[idx_p.at[isl]], pos_buf.at[b], sem_p[b]).wait()

                def row_body(r, c2, _b=b):
                    for g in range(GROUPS):
                        sl = pl.ds(g * LANES, LANES)
                        tok_buf[_b, r, sl] = tok_buf[_b, r, sl] + pos_buf[_b, r, sl]
                    return c2

                lax.fori_loop(0, CHUNK, row_body, 0)
                pltpu.async_copy(tok_buf.at[b], out.at[pl.ds(off, CHUNK)],
                                 sem_o[b])

                nj = j + NBUF

                @pl.when(nj < NCHUNK)
                def _(_b=b, _nj=nj):
                    pltpu.make_async_copy(
                        tok_buf.at[_b],
                        out.at[pl.ds(base + (_nj - NBUF) * CHUNK, CHUNK)],
                        sem_o[_b]).wait()
                    issue(_nj, _b)
            return carry

        lax.fori_loop(0, NCHUNK // NBUF, outer, 0)

        for b in range(NBUF):
            j = NCHUNK - NBUF + b
            pltpu.make_async_copy(
                tok_buf.at[b], out.at[pl.ds(base + j * CHUNK, CHUNK)],
                sem_o[b]).wait()

    return emb_kernel


_EMB_KERNEL = _make_kernel()


def kernel(input_ids, position_ids, token_table, pos_table):
    tok_ids = input_ids.reshape(-1).astype(jnp.int32)
    pos_ids = position_ids.reshape(-1).astype(jnp.int32)
    out = _EMB_KERNEL(tok_ids, pos_ids, token_table, pos_table)
    return out.reshape(BATCH, SEQ, HIDDEN)


# speedup vs baseline: 2.3848x; 1.3022x over previous
import functools

import jax
import jax.numpy as jnp
from jax import lax
from jax.experimental import pallas as pl
from jax.experimental.pallas import tpu as pltpu
from jax.experimental.pallas import tpu_sc as plsc

VOCAB = 100000
HIDDEN = 1024
MAX_POS = 2048
BATCH = 4
SEQ = 2048

_INFO = plsc.get_sparse_core_info()
NC = _INFO.num_cores
NS = _INFO.num_subcores
LANES = _INFO.num_lanes
NW = NC * NS

TOKENS = BATCH * SEQ
TOK_PER_W = TOKENS // NW
CHUNK = 8
NCHUNK = TOK_PER_W // CHUNK
NBUF = 4
GROUPS = HIDDEN // LANES


def _make_kernel():
    mesh = plsc.VectorSubcoreMesh(core_axis_name="c", subcore_axis_name="s")

    @functools.partial(
        pl.kernel,
        mesh=mesh,
        out_type=jax.ShapeDtypeStruct((TOKENS, HIDDEN), jnp.float32),
        scratch_types=[
            pltpu.VMEM((TOK_PER_W,), jnp.int32),
            pltpu.VMEM((TOK_PER_W,), jnp.int32),
            pltpu.VMEM((NBUF, CHUNK, HIDDEN), jnp.float32),
            pltpu.VMEM((NBUF, CHUNK, HIDDEN), jnp.float32),
        ] + [pltpu.SemaphoreType.DMA] * (3 * NBUF),
    )
    def emb_kernel(tok_ids, pos_ids, tok_tab, pos_tab, out,
                   idx_t, idx_p, tok_buf, pos_buf, *sems):
        sem_t = sems[0:NBUF]
        sem_p = sems[NBUF:2 * NBUF]
        sem_o = sems[2 * NBUF:3 * NBUF]
        wid = lax.axis_index("s") * NC + lax.axis_index("c")
        base = wid * TOK_PER_W

        pltpu.sync_copy(tok_ids.at[pl.ds(base, TOK_PER_W)], idx_t)
        pltpu.sync_copy(pos_ids.at[pl.ds(base, TOK_PER_W)], idx_p)

        def issue(j, b):
            isl = pl.ds(j * CHUNK, CHUNK)
            pltpu.async_copy(tok_tab.at[idx_t.at[isl]], tok_buf.at[b], sem_t[b])
            pltpu.async_copy(pos_tab.at[idx_p.at[isl]], pos_buf.at[b], sem_p[b])

        for b in range(NBUF):
            issue(b, b)

        def outer(jj, carry):
            for b in range(NBUF):
                j = jj * NBUF + b
                off = base + j * CHUNK
                isl = pl.ds(j * CHUNK, CHUNK)
                pltpu.make_async_copy(
                    tok_tab.at[idx_t.at[isl]], tok_buf.at[b], sem_t[b]).wait()
                pltpu.make_async_copy(
                    pos_tab.at[idx_p.at[isl]], pos_buf.at[b], sem_p[b]).wait()

                def row_body(r, c2, _b=b):
                    for g in range(GROUPS):
                        sl = pl.ds(g * LANES, LANES)
                        tok_buf[_b, r, sl] = tok_buf[_b, r, sl] + pos_buf[_b, r, sl]
                    return c2

                lax.fori_loop(0, CHUNK, row_body, 0)
                pltpu.async_copy(tok_buf.at[b], out.at[pl.ds(off, CHUNK)],
                                 sem_o[b])

                nj = j + NBUF

                @pl.when(nj < NCHUNK)
                def _(_b=b, _nj=nj):
                    pltpu.make_async_copy(
                        tok_buf.at[_b],
                        out.at[pl.ds(base + (_nj - NBUF) * CHUNK, CHUNK)],
                        sem_o[_b]).wait()
                    issue(_nj, _b)
            return carry

        lax.fori_loop(0, NCHUNK // NBUF, outer, 0)

        for b in range(NBUF):
            j = NCHUNK - NBUF + b
            pltpu.make_async_copy(
                tok_buf.at[b], out.at[pl.ds(base + j * CHUNK, CHUNK)],
                sem_o[b]).wait()

    return emb_kernel


_EMB_KERNEL = _make_kernel()


def kernel(input_ids, position_ids, token_table, pos_table):
    tok_ids = input_ids.reshape(-1).astype(jnp.int32)
    pos_ids = position_ids.reshape(-1).astype(jnp.int32)
    out = _EMB_KERNEL(tok_ids, pos_ids, token_table, pos_table)
    return out.reshape(BATCH, SEQ, HIDDEN)
